# baseline (speedup 1.0000x reference)
import jax
import jax.numpy as jnp
from jax.experimental import pallas as pl
from jax.experimental.pallas import tpu as pltpu

BLOCK_M = 512
LANES = 128


def _vq_kernel(xs_ref, cb_ref, xc2_ref, c2_ref, out_ref):
    xs = xs_ref[...]
    cb = cb_ref[...]
    x2 = xc2_ref[0, 0, :][:, None]
    c2 = c2_ref[0, 0, :]
    n_cb = cb.shape[0]
    m2 = jax.lax.dot_general(
        xs, cb, (((1,), (1,)), ((), ())),
        preferred_element_type=jnp.float32)
    n_chunks = n_cb // LANES
    mn = jnp.full((xs.shape[0], LANES), jnp.inf, dtype=jnp.float32)
    ch = jnp.zeros((xs.shape[0], LANES), dtype=jnp.int32)
    for c in range(n_chunks):
        sl = slice(c * LANES, (c + 1) * LANES)
        d2 = (x2 + c2[None, sl]) + m2[:, sl]
        d = jnp.sqrt(jnp.maximum(d2, 0.0))
        better = d < mn
        mn = jnp.where(better, d, mn)
        ch = jnp.where(better, jnp.int32(c), ch)
    lane = jax.lax.broadcasted_iota(jnp.int32, mn.shape, 1)
    j = ch * LANES + lane
    row_mn = jnp.min(mn, axis=1, keepdims=True)
    idx = jnp.min(jnp.where(mn == row_mn, j, jnp.int32(2**30)), axis=1)
    out_ref[0, 0, :] = idx.astype(jnp.int32)


def kernel(z_e_x, codebook):
    b, t, e = z_e_x.shape
    x = z_e_x.reshape(-1, e)
    mrows = x.shape[0]
    n_cb = codebook.shape[0]
    g = mrows // BLOCK_M
    x2 = jnp.sum(x * x, axis=1).reshape(g, 1, BLOCK_M)
    c2 = jnp.sum(codebook * codebook, axis=1).reshape(1, 1, n_cb)
    xs = -2.0 * x
    out = pl.pallas_call(
        _vq_kernel,
        grid=(g,),
        in_specs=[
            pl.BlockSpec((BLOCK_M, e), lambda i: (i, 0)),
            pl.BlockSpec((n_cb, e), lambda i: (0, 0)),
            pl.BlockSpec((1, 1, BLOCK_M), lambda i: (i, 0, 0)),
            pl.BlockSpec((1, 1, n_cb), lambda i: (0, 0, 0)),
        ],
        out_specs=pl.BlockSpec((1, 1, BLOCK_M), lambda i: (i, 0, 0)),
        out_shape=jax.ShapeDtypeStruct((g, 1, BLOCK_M), jnp.int32),
        compiler_params=pltpu.CompilerParams(
            dimension_semantics=("parallel",)),
    )(xs, codebook, x2, c2)
    return out.reshape(b, t)


# speedup vs baseline: 1.1554x; 1.1554x over previous
import jax
import jax.numpy as jnp
from jax.experimental import pallas as pl
from jax.experimental.pallas import tpu as pltpu

BLOCK_M = 512


def _vq_kernel(xs_ref, cb_ref, x2_ref, c2_ref, out_ref):
    xs = xs_ref[...]
    cb = cb_ref[...]
    x2 = x2_ref[0, 0, :]
    c2 = c2_ref[0, 0, :]
    n_cb = cb.shape[0]
    bm = xs.shape[0]
    m2 = jax.lax.dot_general(
        cb, xs, (((1,), (1,)), ((), ())),
        preferred_element_type=jnp.float32)
    d2 = (c2[:, None] + x2[None, :]) + m2
    d = jnp.sqrt(jnp.maximum(d2, 0.0))
    s = jnp.min(d, axis=0)
    iota = jax.lax.broadcasted_iota(jnp.int32, d.shape, 0)
    idx = jnp.min(jnp.where(d == s[None, :], iota, jnp.int32(n_cb)), axis=0)
    out_ref[0, 0, :] = idx.astype(jnp.int32)


def kernel(z_e_x, codebook):
    b, t, e = z_e_x.shape
    x = z_e_x.reshape(-1, e)
    mrows = x.shape[0]
    n_cb = codebook.shape[0]
    g = mrows // BLOCK_M
    x2 = jnp.sum(x * x, axis=1).reshape(g, 1, BLOCK_M)
    c2 = jnp.sum(codebook * codebook, axis=1).reshape(1, 1, n_cb)
    xs = -2.0 * x
    out = pl.pallas_call(
        _vq_kernel,
        grid=(g,),
        in_specs=[
            pl.BlockSpec((BLOCK_M, e), lambda i: (i, 0)),
            pl.BlockSpec((n_cb, e), lambda i: (0, 0)),
            pl.BlockSpec((1, 1, BLOCK_M), lambda i: (i, 0, 0)),
            pl.BlockSpec((1, 1, n_cb), lambda i: (0, 0, 0)),
        ],
        out_specs=pl.BlockSpec((1, 1, BLOCK_M), lambda i: (i, 0, 0)),
        out_shape=jax.ShapeDtypeStruct((g, 1, BLOCK_M), jnp.int32),
        compiler_params=pltpu.CompilerParams(
            dimension_semantics=("parallel",)),
    )(xs, codebook, x2, c2)
    return out.reshape(b, t)


# speedup vs baseline: 1.3067x; 1.1309x over previous
import jax
import jax.numpy as jnp
from jax.experimental import pallas as pl
from jax.experimental.pallas import tpu as pltpu

BLOCK_M = 512
HALF = 512


def _vq_kernel(x_ref, cb_ref, x2_ref, c2_ref, out_ref):
    xs = -2.0 * x_ref[...]
    x2 = x2_ref[0, 0, :]
    n_cb = cb_ref.shape[0]
    bm = xs.shape[0]

    svals = []
    iotas = []
    ds = []
    for h in range(n_cb // HALF):
        cb_h = cb_ref[pl.ds(h * HALF, HALF), :]
        c2_h = c2_ref[0, 0, pl.ds(h * HALF, HALF)]
        m2 = jax.lax.dot_general(
            cb_h, xs, (((1,), (1,)), ((), ())),
            preferred_element_type=jnp.float32)
        d2 = (c2_h[:, None] + x2[None, :]) + m2
        d = jnp.sqrt(jnp.maximum(d2, 0.0))
        svals.append(jnp.min(d, axis=0))
        iotas.append((jax.lax.broadcasted_iota(jnp.int32, d.shape, 0)
                      + jnp.int32(h * HALF)).astype(jnp.float32))
        ds.append(d)
    s = jnp.minimum(*svals) if len(svals) > 1 else svals[0]
    cands = [jnp.min(jnp.where(d == s[None, :], io, jnp.float32(n_cb)),
                     axis=0)
             for d, io in zip(ds, iotas)]
    idx = jnp.minimum(*cands) if len(cands) > 1 else cands[0]
    out_ref[0, 0, :] = idx.astype(jnp.int32)


def kernel(z_e_x, codebook):
    b, t, e = z_e_x.shape
    x = z_e_x.reshape(-1, e)
    mrows = x.shape[0]
    n_cb = codebook.shape[0]
    g = mrows // BLOCK_M
    x2 = jnp.sum(x * x, axis=1).reshape(g, 1, BLOCK_M)
    c2 = jnp.sum(codebook * codebook, axis=1).reshape(1, 1, n_cb)
    out = pl.pallas_call(
        _vq_kernel,
        grid=(g,),
        in_specs=[
            pl.BlockSpec((BLOCK_M, e), lambda i: (i, 0)),
            pl.BlockSpec((n_cb, e), lambda i: (0, 0)),
            pl.BlockSpec((1, 1, BLOCK_M), lambda i: (i, 0, 0)),
            pl.BlockSpec((1, 1, n_cb), lambda i: (0, 0, 0)),
        ],
        out_specs=pl.BlockSpec((1, 1, BLOCK_M), lambda i: (i, 0, 0)),
        out_shape=jax.ShapeDtypeStruct((g, 1, BLOCK_M), jnp.int32),
        compiler_params=pltpu.CompilerParams(
            dimension_semantics=("parallel",)),
    )(x, codebook, x2, c2)
    return out.reshape(b, t)


# speedup vs baseline: 1.4199x; 1.0867x over previous
import jax
import jax.numpy as jnp
from jax.experimental import pallas as pl
from jax.experimental.pallas import tpu as pltpu

BLOCK_M = 1024
HALF = 512


def _vq_kernel(x_ref, cb_ref, x2_ref, c2_ref, out_ref):
    xs = -2.0 * x_ref[...]
    x2 = x2_ref[0, 0, :]
    n_cb = cb_ref.shape[0]
    bm = xs.shape[0]

    svals = []
    iotas = []
    ds = []
    for h in range(n_cb // HALF):
        cb_h = cb_ref[pl.ds(h * HALF, HALF), :]
        c2_h = c2_ref[0, 0, pl.ds(h * HALF, HALF)]
        m2 = jax.lax.dot_general(
            cb_h, xs, (((1,), (1,)), ((), ())),
            preferred_element_type=jnp.float32)
        d2 = (c2_h[:, None] + x2[None, :]) + m2
        d = jnp.sqrt(jnp.maximum(d2, 0.0))
        svals.append(jnp.min(d, axis=0))
        iotas.append((jax.lax.broadcasted_iota(jnp.int32, d.shape, 0)
                      + jnp.int32(h * HALF)).astype(jnp.float32))
        ds.append(d)
    s = jnp.minimum(*svals) if len(svals) > 1 else svals[0]
    cands = [jnp.min(jnp.where(d == s[None, :], io, jnp.float32(n_cb)),
                     axis=0)
             for d, io in zip(ds, iotas)]
    idx = jnp.minimum(*cands) if len(cands) > 1 else cands[0]
    out_ref[0, 0, :] = idx.astype(jnp.int32)


def kernel(z_e_x, codebook):
    b, t, e = z_e_x.shape
    x = z_e_x.reshape(-1, e)
    mrows = x.shape[0]
    n_cb = codebook.shape[0]
    g = mrows // BLOCK_M
    x2 = jnp.sum(x * x, axis=1).reshape(g, 1, BLOCK_M)
    c2 = jnp.sum(codebook * codebook, axis=1).reshape(1, 1, n_cb)
    out = pl.pallas_call(
        _vq_kernel,
        grid=(g,),
        in_specs=[
            pl.BlockSpec((BLOCK_M, e), lambda i: (i, 0)),
            pl.BlockSpec((n_cb, e), lambda i: (0, 0)),
            pl.BlockSpec((1, 1, BLOCK_M), lambda i: (i, 0, 0)),
            pl.BlockSpec((1, 1, n_cb), lambda i: (0, 0, 0)),
        ],
        out_specs=pl.BlockSpec((1, 1, BLOCK_M), lambda i: (i, 0, 0)),
        out_shape=jax.ShapeDtypeStruct((g, 1, BLOCK_M), jnp.int32),
        compiler_params=pltpu.CompilerParams(
            dimension_semantics=("parallel",)),
    )(x, codebook, x2, c2)
    return out.reshape(b, t)


# speedup vs baseline: 1.4741x; 1.0381x over previous
import jax
import jax.numpy as jnp
from jax.experimental import pallas as pl
from jax.experimental.pallas import tpu as pltpu

BLOCK_M = 2048
HALF = 512


def _vq_kernel(x_ref, cb_ref, x2_ref, c2_ref, out_ref):
    xs = -2.0 * x_ref[...]
    x2 = x2_ref[0, 0, :]
    n_cb = cb_ref.shape[0]
    bm = xs.shape[0]

    svals = []
    iotas = []
    ds = []
    for h in range(n_cb // HALF):
        cb_h = cb_ref[pl.ds(h * HALF, HALF), :]
        c2_h = c2_ref[0, 0, pl.ds(h * HALF, HALF)]
        m2 = jax.lax.dot_general(
            cb_h, xs, (((1,), (1,)), ((), ())),
            preferred_element_type=jnp.float32)
        d2 = (c2_h[:, None] + x2[None, :]) + m2
        d = jnp.sqrt(jnp.maximum(d2, 0.0))
        svals.append(jnp.min(d, axis=0))
        iotas.append((jax.lax.broadcasted_iota(jnp.int32, d.shape, 0)
                      + jnp.int32(h * HALF)).astype(jnp.float32))
        ds.append(d)
    s = jnp.minimum(*svals) if len(svals) > 1 else svals[0]
    cands = [jnp.min(jnp.where(d == s[None, :], io, jnp.float32(n_cb)),
                     axis=0)
             for d, io in zip(ds, iotas)]
    idx = jnp.minimum(*cands) if len(cands) > 1 else cands[0]
    out_ref[0, 0, :] = idx.astype(jnp.int32)


def kernel(z_e_x, codebook):
    b, t, e = z_e_x.shape
    x = z_e_x.reshape(-1, e)
    mrows = x.shape[0]
    n_cb = codebook.shape[0]
    g = mrows // BLOCK_M
    x2 = jnp.sum(x * x, axis=1).reshape(g, 1, BLOCK_M)
    c2 = jnp.sum(codebook * codebook, axis=1).reshape(1, 1, n_cb)
    out = pl.pallas_call(
        _vq_kernel,
        grid=(g,),
        in_specs=[
            pl.BlockSpec((BLOCK_M, e), lambda i: (i, 0)),
            pl.BlockSpec((n_cb, e), lambda i: (0, 0)),
            pl.BlockSpec((1, 1, BLOCK_M), lambda i: (i, 0, 0)),
            pl.BlockSpec((1, 1, n_cb), lambda i: (0, 0, 0)),
        ],
        out_specs=pl.BlockSpec((1, 1, BLOCK_M), lambda i: (i, 0, 0)),
        out_shape=jax.ShapeDtypeStruct((g, 1, BLOCK_M), jnp.int32),
        compiler_params=pltpu.CompilerParams(
            dimension_semantics=("parallel",)),
    )(x, codebook, x2, c2)
    return out.reshape(b, t)
